# baseline (speedup 1.0000x reference)
import jax
import jax.numpy as jnp
from jax.experimental import pallas as pl
from jax.experimental.pallas import tpu as pltpu

_BM = 400
_SLOTS = 3


def _gcn_kernel(adj_hbm, x_ref, h0_ref, w_ref, alpha_ref, out_ref, buf, sems):
    i = pl.program_id(0)
    nsteps = pl.num_programs(0)

    hb = _BM // 2

    def half_copy(step, slot, half):
        return pltpu.make_async_copy(
            adj_hbm.at[pl.ds(step * _BM + half * hb, hb), :],
            buf.at[slot, pl.ds(half * hb, hb), :],
            sems.at[slot, half],
        )

    @pl.when(i == 0)
    def _prefetch():
        for s in range(_SLOTS - 1):
            half_copy(s, s, 0).start()
            half_copy(s, s, 1).start()

    slot = jax.lax.rem(i, _SLOTS)
    a = jax.nn.sigmoid(alpha_ref[...]) * 0.5
    for half in range(2):
        half_copy(i, slot, half).wait()
        rows = pl.ds(half * hb, hb)
        hi = jnp.dot(buf[slot, rows, :], x_ref[...],
                     preferred_element_type=jnp.float32)
        support = (1.0 - a) * hi + a * h0_ref[rows, :]
        out_ref[rows, :] = 0.25 * jnp.dot(
            support, w_ref[...], preferred_element_type=jnp.float32
        ) + 0.75 * support

    nxt = i + _SLOTS - 1
    @pl.when(nxt < nsteps)
    def _issue_next():
        nslot = jax.lax.rem(nxt, _SLOTS)
        half_copy(nxt, nslot, 0).start()
        half_copy(nxt, nslot, 1).start()


def kernel(input, adj, h0, lamda, l, weight, alpha):
    del lamda, l
    n, d_in = input.shape
    alpha2d = alpha.reshape(1, 1)
    return pl.pallas_call(
        _gcn_kernel,
        grid=(n // _BM,),
        in_specs=[
            pl.BlockSpec(memory_space=pltpu.MemorySpace.HBM),
            pl.BlockSpec((n, d_in), lambda i: (0, 0)),
            pl.BlockSpec((_BM, d_in), lambda i: (i, 0)),
            pl.BlockSpec(weight.shape, lambda i: (0, 0)),
            pl.BlockSpec((1, 1), lambda i: (0, 0)),
        ],
        out_specs=pl.BlockSpec((_BM, d_in), lambda i: (i, 0)),
        out_shape=jax.ShapeDtypeStruct((n, weight.shape[1]), jnp.float32),
        scratch_shapes=[
            pltpu.VMEM((_SLOTS, _BM, n), jnp.float32),
            pltpu.SemaphoreType.DMA((_SLOTS, 2)),
        ],
        compiler_params=pltpu.CompilerParams(
            dimension_semantics=("arbitrary",),
        ),
    )(adj, input, h0, weight, alpha2d)


# speedup vs baseline: 1.0864x; 1.0864x over previous
import jax
import jax.numpy as jnp
from jax.experimental import pallas as pl
from jax.experimental.pallas import tpu as pltpu

_BM = 400
_SLOTS = 3


def _gcn_kernel(adj_hbm, x_ref, h0_ref, w_ref, alpha_ref, out_ref, buf, sems):
    i = pl.program_id(0)
    nsteps = pl.num_programs(0)

    def slab_copy(step, slot):
        return pltpu.make_async_copy(
            adj_hbm.at[pl.ds(step * _BM, _BM), :], buf.at[slot], sems.at[slot]
        )

    @pl.when(i == 0)
    def _prefetch():
        for s in range(_SLOTS - 1):
            slab_copy(s, s).start()

    slot = jax.lax.rem(i, _SLOTS)
    slab_copy(i, slot).wait()

    hi = jnp.dot(buf[slot], x_ref[...], preferred_element_type=jnp.float32)
    a = jax.nn.sigmoid(alpha_ref[...]) * 0.5
    support = (1.0 - a) * hi + a * h0_ref[...]
    out_ref[...] = 0.25 * jnp.dot(
        support, w_ref[...], preferred_element_type=jnp.float32
    ) + 0.75 * support

    nxt = i + _SLOTS - 1
    @pl.when(nxt < nsteps)
    def _issue_next():
        slab_copy(nxt, jax.lax.rem(nxt, _SLOTS)).start()


def kernel(input, adj, h0, lamda, l, weight, alpha):
    del lamda, l
    n, d_in = input.shape
    alpha2d = alpha.reshape(1, 1)
    return pl.pallas_call(
        _gcn_kernel,
        grid=(n // _BM,),
        in_specs=[
            pl.BlockSpec(memory_space=pltpu.MemorySpace.HBM),
            pl.BlockSpec((n, d_in), lambda i: (0, 0)),
            pl.BlockSpec((_BM, d_in), lambda i: (i, 0)),
            pl.BlockSpec(weight.shape, lambda i: (0, 0)),
            pl.BlockSpec((1, 1), lambda i: (0, 0)),
        ],
        out_specs=pl.BlockSpec((_BM, d_in), lambda i: (i, 0)),
        out_shape=jax.ShapeDtypeStruct((n, weight.shape[1]), jnp.float32),
        scratch_shapes=[
            pltpu.VMEM((_SLOTS, _BM, n), jnp.float32),
            pltpu.SemaphoreType.DMA((_SLOTS,)),
        ],
        compiler_params=pltpu.CompilerParams(
            dimension_semantics=("arbitrary",),
        ),
    )(adj, input, h0, weight, alpha2d)
